# baseline (speedup 1.0000x reference)
import functools

import jax
import jax.numpy as jnp
from jax import lax
from jax.experimental import pallas as pl
from jax.experimental.pallas import tpu as pltpu
from jax.experimental.pallas import tpu_sc as plsc

NEG = -1e30
MAX_ANS_LEN = 30
B, S, H = 2, 2048, 1024
BS = 2048
NB = S // BS
NTILES = 16
CHUNK = S // NTILES
WPAD = 128
NVREG = CHUNK // 16
INT_MAX = 2**31 - 1


def _proj_body(x_ref, m_ref, w_ref, b_ref, s_ref, e_ref):
    x = x_ref[0]
    w = w_ref[...]
    se = lax.dot_general(x, w, (((1,), (0,)), ((), ())),
                         preferred_element_type=jnp.float32)
    m = m_ref[0, 0]
    s = se[:, 0] + b_ref[0]
    e = se[:, 1] + b_ref[1]
    valid = m > 0
    s_ref[0, 0] = jnp.where(valid, s, NEG)
    e_ref[0, 0] = jnp.where(valid, e, NEG)


_proj_call = pl.pallas_call(
    _proj_body,
    grid=(B * NB,),
    in_specs=[
        pl.BlockSpec((1, BS, H), lambda k: (k, 0, 0)),
        pl.BlockSpec((1, 1, BS), lambda k: (k, 0, 0)),
        pl.BlockSpec((H, 2), lambda k: (0, 0)),
        pl.BlockSpec(memory_space=pltpu.SMEM),
    ],
    out_specs=[
        pl.BlockSpec((1, 1, BS), lambda k: (k, 0, 0)),
        pl.BlockSpec((1, 1, BS), lambda k: (k, 0, 0)),
    ],
    out_shape=[
        jax.ShapeDtypeStruct((B * NB, 1, BS), jnp.float32),
        jax.ShapeDtypeStruct((B * NB, 1, BS), jnp.float32),
    ],
)


_mesh = plsc.VectorSubcoreMesh(core_axis_name="c", subcore_axis_name="s")


@functools.partial(
    pl.kernel,
    mesh=_mesh,
    compiler_params=pltpu.CompilerParams(needs_layout_passes=False),
    out_type=[
        jax.ShapeDtypeStruct((B, NTILES, 16), jnp.float32),
        jax.ShapeDtypeStruct((B, NTILES, 16), jnp.int32),
        jax.ShapeDtypeStruct((B, 1, 16), jnp.int32),
        jax.ShapeDtypeStruct((B, 1, 16), jnp.int32),
    ],
    scratch_types=[
        pltpu.VMEM((CHUNK,), jnp.float32),
        pltpu.VMEM((CHUNK + WPAD,), jnp.float32),
        pltpu.VMEM((16,), jnp.float32),
        pltpu.VMEM((16,), jnp.int32),
        pltpu.VMEM((NTILES, 16), jnp.float32),
        pltpu.VMEM((NTILES, 16), jnp.int32),
        pltpu.VMEM((16,), jnp.int32),
        pltpu.VMEM((16,), jnp.int32),
    ],
)
def _span_topk(s_hbm, e_hbm, stv_hbm, stf_hbm, oi_hbm, oj_hbm,
               s_v, e_v, rowv, rowf, gbv, gbf, oiv, ojv):
    c = lax.axis_index("c")
    t = lax.axis_index("s")
    base = t * CHUNK
    pltpu.sync_copy(s_hbm.at[c, 0, pl.ds(base, CHUNK)], s_v)

    @pl.when(t < NTILES - 1)
    def _load_full():
        pltpu.sync_copy(e_hbm.at[c, 0, pl.ds(base, CHUNK + WPAD)], e_v)

    @pl.when(t == NTILES - 1)
    def _load_last():
        pltpu.sync_copy(e_hbm.at[c, 0, pl.ds(base, CHUNK)],
                        e_v.at[pl.ds(0, CHUNK)])
        negv = lax.broadcast(jnp.float32(NEG), (16,))
        e_v[pl.ds(CHUNK, 16)] = negv
        e_v[pl.ds(CHUNK + 16, 16)] = negv

    lane = lax.iota(jnp.int32, 16)
    bval = None
    for v in range(NVREG):
        i0 = v * 16
        sv = s_v[pl.ds(i0, 16)]
        m = e_v[pl.ds(i0, 16)]
        db = lax.broadcast(jnp.int32(0), (16,))
        for d in range(1, MAX_ANS_LEN):
            cand = e_v[pl.ds(i0 + d, 16)]
            gt = cand > m
            m = jnp.where(gt, cand, m)
            db = jnp.where(gt, jnp.int32(d), db)
        score = sv + m
        i_vec = lane + (base + i0)
        flat = i_vec * S + (i_vec + db)
        if v == 0:
            ovr = jnp.logical_and(lax.broadcast(t == 0, (16,)), lane < 4)
            diag_ok = jnp.logical_and(lane >= 1, lane < 4)
            diag_score = jnp.where(diag_ok, sv + e_v[pl.ds(0, 16)],
                                   jnp.float32(NEG))
            score = jnp.where(ovr, diag_score, score)
            flat = jnp.where(ovr, i_vec * (S + 1), flat)
            bval, bflat = score, flat
        else:
            gt = score > bval
            bval = jnp.where(gt, score, bval)
            bflat = jnp.where(gt, flat, bflat)

    rowv[...] = bval
    rowf[...] = bflat
    pltpu.sync_copy(rowv, stv_hbm.at[c, t])
    pltpu.sync_copy(rowf, stf_hbm.at[c, t])
    plsc.subcore_barrier()

    @pl.when(t == 0)
    def _merge():
        pltpu.sync_copy(stv_hbm.at[c], gbv)
        pltpu.sync_copy(stf_hbm.at[c], gbf)
        accv = gbv[0]
        accf = gbf[0]
        for r in range(1, NTILES):
            rv = gbv[r]
            rf = gbf[r]
            take = jnp.logical_or(
                rv > accv, jnp.logical_and(rv == accv, rf < accf))
            accv = jnp.where(take, rv, accv)
            accf = jnp.where(take, rf, accf)
        mv = jnp.max(accv)
        fl = jnp.min(jnp.where(accv == mv, accf, jnp.int32(INT_MAX)))
        oiv[...] = lax.broadcast(fl // S, (16,))
        ojv[...] = lax.broadcast(fl % S, (16,))
        pltpu.sync_copy(oiv, oi_hbm.at[c, 0])
        pltpu.sync_copy(ojv, oj_hbm.at[c, 0])


def kernel(seq_hiddens, ans_mask, top_k, W, b):
    del top_k
    x4 = seq_hiddens.reshape(B * NB, BS, H)
    m4 = ans_mask.reshape(B * NB, 1, BS)
    s3, e3 = _proj_call(x4, m4, W, b)
    start_logits = s3.reshape(B, S)
    end_logits = e3.reshape(B, S)
    _, _, oi, oj = _span_topk(start_logits.reshape(B, 1, S),
                              end_logits.reshape(B, 1, S))
    return (start_logits, end_logits, oi[:, 0, 0], oj[:, 0, 0])


# speedup vs baseline: 1.0208x; 1.0208x over previous
import functools

import jax
import jax.numpy as jnp
from jax import lax
from jax.experimental import pallas as pl
from jax.experimental.pallas import tpu as pltpu
from jax.experimental.pallas import tpu_sc as plsc

NEG = -1e30
MAX_ANS_LEN = 30
B, S, H = 2, 2048, 1024
BS = 2048
NB = S // BS
NTILES = 16
CHUNK = S // NTILES
WPAD = 128
NVREG = CHUNK // 16
INT_MAX = 2**31 - 1


def _proj_body(x_ref, m_ref, w_ref, b_ref, s_ref, e_ref):
    x = x_ref[0]
    w = w_ref[...]
    se = lax.dot_general(x, w, (((1,), (0,)), ((), ())),
                         preferred_element_type=jnp.float32)
    m = m_ref[0, 0]
    s = se[:, 0] + b_ref[0]
    e = se[:, 1] + b_ref[1]
    valid = m > 0
    s_ref[0, 0] = jnp.where(valid, s, NEG)
    e_ref[0, 0] = jnp.where(valid, e, NEG)


_proj_call = pl.pallas_call(
    _proj_body,
    grid=(B * NB,),
    in_specs=[
        pl.BlockSpec((1, BS, H), lambda k: (k, 0, 0)),
        pl.BlockSpec((1, 1, BS), lambda k: (k, 0, 0)),
        pl.BlockSpec((H, 2), lambda k: (0, 0)),
        pl.BlockSpec(memory_space=pltpu.SMEM),
    ],
    out_specs=[
        pl.BlockSpec((1, 1, BS), lambda k: (k, 0, 0)),
        pl.BlockSpec((1, 1, BS), lambda k: (k, 0, 0)),
    ],
    out_shape=[
        jax.ShapeDtypeStruct((B * NB, 1, BS), jnp.float32),
        jax.ShapeDtypeStruct((B * NB, 1, BS), jnp.float32),
    ],
)


_mesh = plsc.VectorSubcoreMesh(core_axis_name="c", subcore_axis_name="s")


@functools.partial(
    pl.kernel,
    mesh=_mesh,
    compiler_params=pltpu.CompilerParams(needs_layout_passes=False),
    out_type=[
        jax.ShapeDtypeStruct((B, NTILES, 32), jnp.int32),
        jax.ShapeDtypeStruct((B, 1, 32), jnp.int32),
    ],
    scratch_types=[
        pltpu.VMEM((CHUNK,), jnp.float32),
        pltpu.VMEM((CHUNK + WPAD,), jnp.float32),
        pltpu.VMEM((32,), jnp.int32),
        pltpu.VMEM((NTILES, 32), jnp.int32),
        pltpu.VMEM((32,), jnp.int32),
    ],
)
def _span_topk(s_hbm, e_hbm, st_hbm, o_hbm,
               s_v, e_v, rowb, gb, ob):
    c = lax.axis_index("c")
    t = lax.axis_index("s")
    base = t * CHUNK
    pltpu.sync_copy(s_hbm.at[c, 0, pl.ds(base, CHUNK)], s_v)

    @pl.when(t < NTILES - 1)
    def _load_full():
        pltpu.sync_copy(e_hbm.at[c, 0, pl.ds(base, CHUNK + WPAD)], e_v)

    @pl.when(t == NTILES - 1)
    def _load_last():
        pltpu.sync_copy(e_hbm.at[c, 0, pl.ds(base, CHUNK)],
                        e_v.at[pl.ds(0, CHUNK)])
        negv = lax.broadcast(jnp.float32(NEG), (16,))
        e_v[pl.ds(CHUNK, 16)] = negv
        e_v[pl.ds(CHUNK + 16, 16)] = negv

    lane = lax.iota(jnp.int32, 16)
    bval = None
    for v in range(NVREG):
        i0 = v * 16
        sv = s_v[pl.ds(i0, 16)]
        m = e_v[pl.ds(i0, 16)]
        db = lax.broadcast(jnp.int32(0), (16,))
        for d in range(1, MAX_ANS_LEN):
            cand = e_v[pl.ds(i0 + d, 16)]
            gt = cand > m
            m = jnp.where(gt, cand, m)
            db = jnp.where(gt, jnp.int32(d), db)
        score = sv + m
        i_vec = lane + (base + i0)
        flat = i_vec * S + (i_vec + db)
        if v == 0:
            ovr = jnp.logical_and(lax.broadcast(t == 0, (16,)), lane < 4)
            diag_ok = jnp.logical_and(lane >= 1, lane < 4)
            diag_score = jnp.where(diag_ok, sv + e_v[pl.ds(0, 16)],
                                   jnp.float32(NEG))
            score = jnp.where(ovr, diag_score, score)
            flat = jnp.where(ovr, i_vec * (S + 1), flat)
            bval, bflat = score, flat
        else:
            gt = score > bval
            bval = jnp.where(gt, score, bval)
            bflat = jnp.where(gt, flat, bflat)

    rowb[pl.ds(0, 16)] = plsc.bitcast(bval, jnp.int32)
    rowb[pl.ds(16, 16)] = bflat
    pltpu.sync_copy(rowb, st_hbm.at[c, t])
    plsc.subcore_barrier()

    @pl.when(t == 0)
    def _merge():
        pltpu.sync_copy(st_hbm.at[c], gb)
        accv = plsc.bitcast(gb[0, pl.ds(0, 16)], jnp.float32)
        accf = gb[0, pl.ds(16, 16)]
        for r in range(1, NTILES):
            rv = plsc.bitcast(gb[r, pl.ds(0, 16)], jnp.float32)
            rf = gb[r, pl.ds(16, 16)]
            take = jnp.logical_or(
                rv > accv, jnp.logical_and(rv == accv, rf < accf))
            accv = jnp.where(take, rv, accv)
            accf = jnp.where(take, rf, accf)
        mv = jnp.max(accv)
        fl = jnp.min(jnp.where(accv == mv, accf, jnp.int32(INT_MAX)))
        ob[pl.ds(0, 16)] = lax.broadcast(fl // S, (16,))
        ob[pl.ds(16, 16)] = lax.broadcast(fl % S, (16,))
        pltpu.sync_copy(ob, o_hbm.at[c, 0])


def kernel(seq_hiddens, ans_mask, top_k, W, b):
    del top_k
    x4 = seq_hiddens.reshape(B * NB, BS, H)
    m4 = ans_mask.reshape(B * NB, 1, BS)
    s3, e3 = _proj_call(x4, m4, W, b)
    start_logits = s3.reshape(B, S)
    end_logits = e3.reshape(B, S)
    _, o = _span_topk(start_logits.reshape(B, 1, S),
                      end_logits.reshape(B, 1, S))
    return (start_logits, end_logits, o[:, 0, 0], o[:, 0, 16])
